# Initial kernel scaffold; baseline (speedup 1.0000x reference)
import jax
import jax.numpy as jnp
from jax.experimental import pallas as pl


def kernel(x, token_embedding):
    raise NotImplementedError("write your pallas kernel here")


# speedup vs baseline: 1.0195x; 1.0195x over previous
import functools

import jax
import jax.numpy as jnp
from jax import lax
from jax.experimental import pallas as pl
from jax.experimental.pallas import tpu as pltpu
from jax.experimental.pallas import tpu_sc as plsc

VOCAB = 1000
D = 1000
B_TOTAL = 1024 * 50
NC, NS = 2, 16
NW = NC * NS
BPW = B_TOTAL // NW
CHUNK = 80
NCHUNK = BPW // CHUNK

_mesh = plsc.VectorSubcoreMesh(core_axis_name="c", subcore_axis_name="s")


@functools.partial(
    pl.kernel,
    mesh=_mesh,
    out_type=jax.ShapeDtypeStruct((B_TOTAL, D), jnp.float32),
    scratch_types=[
        pltpu.VMEM((BPW,), jnp.int32),
        pltpu.VMEM((CHUNK, D), jnp.float32),
        pltpu.SemaphoreType.DMA,
    ],
    compiler_params=pltpu.CompilerParams(use_tc_tiling_on_sc=False),
)
def _gather_all(x_hbm, table_hbm, out_hbm, idx_v, rows_v, sem):
    wid = lax.axis_index("s") * NC + lax.axis_index("c")
    base = wid * BPW
    pltpu.sync_copy(x_hbm.at[pl.ds(base, BPW)], idx_v)

    def body(i, _):
        off = pl.multiple_of(i * CHUNK, 8)
        pltpu.async_copy(
            table_hbm.at[idx_v.at[pl.ds(off, CHUNK)]], rows_v, sem
        ).wait()
        pltpu.sync_copy(rows_v, out_hbm.at[pl.ds(base + off, CHUNK)])
        return ()

    lax.fori_loop(0, NCHUNK, body, ())


def kernel(x, token_embedding):
    flat = x.reshape(-1).astype(jnp.int32)
    out = _gather_all(flat, token_embedding)
    return out.reshape(x.shape[0], x.shape[1], VOCAB)


# speedup vs baseline: 1.0339x; 1.0142x over previous
import functools

import jax
import jax.numpy as jnp
from jax import lax
from jax.experimental import pallas as pl
from jax.experimental.pallas import tpu as pltpu
from jax.experimental.pallas import tpu_sc as plsc

VOCAB = 1000
D = 1000
B_TOTAL = 1024 * 50
NC, NS = 2, 16
NW = NC * NS
BPW = B_TOTAL // NW
CHUNK = 40
NCHUNK = BPW // CHUNK
NJ = NCHUNK // 2

_mesh = plsc.VectorSubcoreMesh(core_axis_name="c", subcore_axis_name="s")


@functools.partial(
    pl.kernel,
    mesh=_mesh,
    out_type=jax.ShapeDtypeStruct((B_TOTAL, D), jnp.float32),
    scratch_types=[
        pltpu.VMEM((BPW,), jnp.int32),
        pltpu.VMEM((CHUNK, D), jnp.float32),
        pltpu.VMEM((CHUNK, D), jnp.float32),
        pltpu.SemaphoreType.DMA,
        pltpu.SemaphoreType.DMA,
    ],
    compiler_params=pltpu.CompilerParams(use_tc_tiling_on_sc=False),
)
def _gather_all(x_hbm, table_hbm, out_hbm, idx_v, rows0, rows1, gsem, ssem):
    wid = lax.axis_index("s") * NC + lax.axis_index("c")
    base = wid * BPW
    pltpu.sync_copy(x_hbm.at[pl.ds(base, BPW)], idx_v)

    def gather(i, buf):
        off = pl.multiple_of(i * CHUNK, 8)
        return pltpu.make_async_copy(
            table_hbm.at[idx_v.at[pl.ds(off, CHUNK)]], buf, gsem
        )

    def scatter(i, buf):
        off = pl.multiple_of(i * CHUNK, 8)
        return pltpu.make_async_copy(buf, out_hbm.at[pl.ds(base + off, CHUNK)], ssem)

    gather(0, rows0).start()

    def body(j, _):
        i0 = j * 2

        @pl.when(j >= 1)
        def _():
            scatter(i0 - 1, rows1).wait()

        gather(i0 + 1, rows1).start()
        gather(i0, rows0).wait()
        scatter(i0, rows0).start()
        scatter(i0, rows0).wait()

        @pl.when(j < NJ - 1)
        def _():
            gather(i0 + 2, rows0).start()

        gather(i0 + 1, rows1).wait()
        scatter(i0 + 1, rows1).start()
        return ()

    lax.fori_loop(0, NJ, body, ())
    scatter(NCHUNK - 1, rows1).wait()


def kernel(x, token_embedding):
    flat = x.reshape(-1).astype(jnp.int32)
    out = _gather_all(flat, token_embedding)
    return out.reshape(x.shape[0], x.shape[1], VOCAB)


# speedup vs baseline: 1.1490x; 1.1113x over previous
import functools

import jax
import jax.numpy as jnp
from jax import lax
from jax.experimental import pallas as pl
from jax.experimental.pallas import tpu as pltpu
from jax.experimental.pallas import tpu_sc as plsc

VOCAB = 1000
D = 1000
B_TOTAL = 1024 * 50
NC, NS = 2, 16
NW = NC * NS
BPW = B_TOTAL // NW
CHUNK = 32
NCHUNK = BPW // CHUNK
NJ = NCHUNK // 2

_mesh = plsc.VectorSubcoreMesh(core_axis_name="c", subcore_axis_name="s")


@functools.partial(
    pl.kernel,
    mesh=_mesh,
    out_type=jax.ShapeDtypeStruct((B_TOTAL, D), jnp.float32),
    scratch_types=[
        pltpu.VMEM_SHARED((VOCAB, D), jnp.float32),
        pltpu.VMEM((BPW,), jnp.int32),
        pltpu.VMEM((CHUNK, D), jnp.float32),
        pltpu.VMEM((CHUNK, D), jnp.float32),
        pltpu.SemaphoreType.DMA,
        pltpu.SemaphoreType.DMA,
    ],
    compiler_params=pltpu.CompilerParams(use_tc_tiling_on_sc=False),
)
def _gather_all(x_hbm, table_hbm, out_hbm, table_sp, idx_v, rows0, rows1,
                gsem, ssem):
    cid = lax.axis_index("c")
    sid = lax.axis_index("s")
    wid = sid * NC + cid
    base = wid * BPW

    @pl.when(sid == 0)
    def _():
        pltpu.sync_copy(table_hbm, table_sp)

    pltpu.sync_copy(x_hbm.at[pl.ds(base, BPW)], idx_v)
    plsc.subcore_barrier()

    def gather(i, buf):
        off = pl.multiple_of(i * CHUNK, 8)
        return pltpu.make_async_copy(
            table_sp.at[idx_v.at[pl.ds(off, CHUNK)]], buf, gsem
        )

    def scatter(i, buf):
        off = pl.multiple_of(i * CHUNK, 8)
        return pltpu.make_async_copy(buf, out_hbm.at[pl.ds(base + off, CHUNK)], ssem)

    gather(0, rows0).start()

    def body(j, _):
        i0 = j * 2

        @pl.when(j >= 1)
        def _():
            scatter(i0 - 1, rows1).wait()

        gather(i0 + 1, rows1).start()
        gather(i0, rows0).wait()
        scatter(i0, rows0).start()
        scatter(i0, rows0).wait()

        @pl.when(j < NJ - 1)
        def _():
            gather(i0 + 2, rows0).start()

        gather(i0 + 1, rows1).wait()
        scatter(i0 + 1, rows1).start()
        return ()

    lax.fori_loop(0, NJ, body, ())
    scatter(NCHUNK - 1, rows1).wait()


def kernel(x, token_embedding):
    flat = x.reshape(-1).astype(jnp.int32)
    out = _gather_all(flat, token_embedding)
    return out.reshape(x.shape[0], x.shape[1], VOCAB)
